# baseline (speedup 1.0000x reference)
import functools

import jax
import jax.numpy as jnp
from jax import lax
from jax.experimental import pallas as pl
from jax.experimental.pallas import tpu as pltpu
from jax.experimental.pallas import tpu_sc as plsc

N = 10000
E = 320000
B = 100
EXPR = 2048
DF = 128
DN = 200
S_BN = 1.0 / (1.0 + 1e-5) ** 0.5

NC, NS = 2, 16
NW = NC * NS
CHUNK = 128
CH0 = 160
CHP = CH0 // 4
NCHROW = NS * CH0
STRIPE = 632
NAGG = NS * STRIPE

RB = 2000
R = 200
NBR = N // R
GP = 104

def _sc_agg_body(drug_hbm, src_hbm, dst_hbm, zeros_hbm, out_hbm,
                 src_v, dst_v, rows_v, agg_sp, sem0, sem1):
    c = lax.axis_index("c")
    s = lax.axis_index("s")
    row0 = s * STRIPE
    pltpu.sync_copy(zeros_hbm, agg_sp.at[pl.ds(row0, STRIPE)])
    plsc.subcore_barrier()

    sems = (sem0, sem1)
    tb = s * CH0

    @pl.when(c == 0)
    def _pipeline():
        for phase in range(4):
            base = pl.multiple_of(tb + phase * CHP, 8)
            pltpu.sync_copy(src_hbm.at[pl.ds(base, CHP)], src_v)
            pltpu.sync_copy(dst_hbm.at[pl.ds(base, CHP)], dst_v)
            pltpu.async_copy(drug_hbm.at[src_v.at[0]], rows_v.at[0], sem0)
            pltpu.async_copy(drug_hbm.at[src_v.at[1]], rows_v.at[1], sem1)

            def pair_body(g, carry):
                for b in range(2):
                    j = 2 * g + b
                    pltpu.make_async_copy(drug_hbm.at[pl.ds(0, CHUNK)],
                                          rows_v.at[b], sems[b]).wait()

                    @pl.when(j + 2 < CHP)
                    def _prefetch(j=j, b=b):
                        pltpu.async_copy(drug_hbm.at[src_v.at[j + 2]],
                                         rows_v.at[b], sems[b])

                    pltpu.sync_copy(rows_v.at[b], agg_sp.at[dst_v.at[j]],
                                    add=True)
                return carry

            lax.fori_loop(0, CHP // 2, pair_body, 0)

    plsc.subcore_barrier()
    pltpu.sync_copy(agg_sp.at[pl.ds(row0, STRIPE)],
                    out_hbm.at[c, pl.ds(row0, STRIPE)])


@functools.lru_cache(maxsize=1)
def _make_sc_agg():
    mesh = plsc.VectorSubcoreMesh(core_axis_name="c", subcore_axis_name="s",
                                  num_cores=NC, num_subcores=NS)
    return pl.kernel(
        _sc_agg_body,
        out_type=jax.ShapeDtypeStruct((NC, NAGG, DF), jnp.float32),
        mesh=mesh,
        scratch_types=[
            pltpu.VMEM((CHP, CHUNK), jnp.int32),
            pltpu.VMEM((CHP, CHUNK), jnp.int32),
            pltpu.VMEM((2, CHUNK, DF), jnp.float32),
            pltpu.VMEM_SHARED((NAGG, DF), jnp.float32),
            pltpu.SemaphoreType.DMA,
            pltpu.SemaphoreType.DMA,
        ],
    )


def _a_body(agg_ref, dx_ref, wrel_ref, brel_ref, wroot_ref, tw_ref,
            h_ref, sc_ref):
    agg = agg_ref[0] + agg_ref[1]
    h = jnp.dot(agg, wrel_ref[...], preferred_element_type=jnp.float32)
    h = h + brel_ref[...]
    h = h + jnp.dot(dx_ref[...], wroot_ref[...],
                    preferred_element_type=jnp.float32)
    h = jnp.maximum(h * S_BN, 0.0)
    h_ref[...] = h
    tw = tw_ref[...]
    inv = lax.rsqrt(jnp.sum(tw * tw))
    sc_ref[...] = jnp.tanh(
        jnp.dot(h, tw, preferred_element_type=jnp.float32) * inv)


def _b_body(jlo_ref, jhi_ref, sr_ref, br_ref, ir_ref,
            sc3_ref, bc3_ref, ic3_ref, mask_ref):
    i = pl.program_id(0)
    sr = sr_ref[...]
    br = br_ref[...]
    ir = ir_ref[...]
    zero = jnp.zeros((R, 1), jnp.float32)

    def col(j, carry):
        rank, n = carry
        scj = sc3_ref[j]
        bcj = bc3_ref[j]
        icj = ic3_ref[j]
        same = br == bcj
        better = (scj > sr) | ((scj == sr) & (icj < ir))
        rank = rank + jnp.sum(jnp.where(same & better, 1.0, 0.0),
                              axis=1, keepdims=True)
        n = n + jnp.sum(jnp.where(same, 1.0, 0.0), axis=1, keepdims=True)
        return rank, n

    rank, n = lax.fori_loop(jlo_ref[i], jhi_ref[i] + 1, col, (zero, zero))
    kper = jnp.ceil(0.8 * n)
    mask_ref[...] = jnp.where(rank < kper, 1.0, 0.0)


def _c_body(blo_ref, bhi_ref, h_ref, sc_ref, mk_ref, br_ref, out_ref,
            cnt_ref, sum1_ref, max1_ref, sum2_ref, max2_ref):
    i = pl.program_id(0)

    @pl.when(i == 0)
    def _init():
        cnt_ref[...] = jnp.zeros((GP, 1), jnp.float32)
        sum1_ref[...] = jnp.zeros((GP, DN), jnp.float32)
        max1_ref[...] = jnp.full((GP, DN), -1e30, jnp.float32)
        sum2_ref[...] = jnp.zeros((GP, DN), jnp.float32)
        max2_ref[...] = jnp.full((GP, DN), -1e30, jnp.float32)

    hb = h_ref[...]
    scb = sc_ref[...]
    mkb = mk_ref[...]
    bb = br_ref[...]
    scaled = hb * scb
    g_lo = blo_ref[i]
    g_hi = bhi_ref[i]

    def per_g(g, carry):
        sel = bb == g
        s1 = jnp.sum(jnp.where(sel, hb, 0.0), axis=0, keepdims=True)
        m1 = jnp.max(jnp.where(sel, hb, -1e30), axis=0, keepdims=True)
        selm = sel & (mkb > 0.5)
        s2 = jnp.sum(jnp.where(selm, scaled, 0.0), axis=0, keepdims=True)
        m2 = jnp.max(jnp.where(selm, scaled, -1e30), axis=0, keepdims=True)
        cg = jnp.sum(jnp.where(sel, 1.0, 0.0), axis=0, keepdims=True)
        cnt_ref[pl.ds(g, 1), :] = cnt_ref[pl.ds(g, 1), :] + cg
        sum1_ref[pl.ds(g, 1), :] = sum1_ref[pl.ds(g, 1), :] + s1
        max1_ref[pl.ds(g, 1), :] = jnp.maximum(max1_ref[pl.ds(g, 1), :], m1)
        sum2_ref[pl.ds(g, 1), :] = sum2_ref[pl.ds(g, 1), :] + s2
        max2_ref[pl.ds(g, 1), :] = jnp.maximum(max2_ref[pl.ds(g, 1), :], m2)
        return carry

    lax.fori_loop(g_lo, g_hi + 1, per_g, 0)

    @pl.when(i == NBR - 1)
    def _fin():
        cnt = cnt_ref[...]
        kper = jnp.ceil(0.8 * cnt)
        gmp1 = jnp.where(cnt > 0, max1_ref[...], 0.0)
        gap1 = sum1_ref[...] / jnp.maximum(cnt, 1.0)
        gmp2 = jnp.where(kper > 0, max2_ref[...], 0.0)
        gap2 = sum2_ref[...] / jnp.maximum(kper, 1.0)
        out_ref[0] = jnp.maximum(gmp1 + 3.0 * gmp2, 0.0)
        out_ref[1] = jnp.maximum(gap1 + 3.0 * gap2, 0.0)


def _d_body(x_ref, w1_ref, b1_ref, w2_ref, b2_ref, ax_ref,
            pwf_ref, pwa_ref, pb_ref, f_ref, r_ref):
    x = x_ref[...] * S_BN
    h1 = jnp.dot(x, w1_ref[...], preferred_element_type=jnp.float32)
    h1 = jnp.maximum((h1 + b1_ref[...]) * S_BN, 0.0)
    f = jnp.dot(h1, w2_ref[...], preferred_element_type=jnp.float32)
    f = jnp.maximum((f + b2_ref[...]) * S_BN, 0.0)
    f_ref[...] = f
    f1 = f[0:B, :]
    r = jnp.dot(f1, pwf_ref[...], preferred_element_type=jnp.float32)
    r = r + jnp.dot(ax_ref[0], pwa_ref[0],
                    preferred_element_type=jnp.float32)
    r = r + jnp.dot(ax_ref[1], pwa_ref[1],
                    preferred_element_type=jnp.float32)
    r_ref[...] = r + pb_ref[...]


def kernel(x1, x2, batch_idx, edge_attr, edge_index, drug_x,
           em_w1, em_b1, em_w2, em_b2, gc_w_rel, gc_b_rel, gc_w_root,
           topk_w, pm_w, pm_b):
    del edge_attr
    batch_idx = batch_idx.astype(jnp.int32)
    edge_index = edge_index.astype(jnp.int32)

    npad = NCHROW * CHUNK - E
    src_p = jnp.concatenate(
        [edge_index[0], jnp.full((npad,), N, jnp.int32)]
    ).reshape(NCHROW, CHUNK)
    dst_p = jnp.concatenate(
        [edge_index[1], jnp.zeros((npad,), jnp.int32)]
    ).reshape(NCHROW, CHUNK)
    drug_pad = jnp.concatenate(
        [drug_x, jnp.zeros((16, DF), jnp.float32)], axis=0)
    zeros_hbm = jnp.zeros((STRIPE, DF), jnp.float32)
    agg = _make_sc_agg()(drug_pad, src_p, dst_p, zeros_hbm)[:, :N, :]

    h, score = pl.pallas_call(
        _a_body,
        grid=(N // RB,),
        in_specs=[
            pl.BlockSpec((NC, RB, DF), lambda i: (0, i, 0)),
            pl.BlockSpec((RB, DF), lambda i: (i, 0)),
            pl.BlockSpec((DF, DN), lambda i: (0, 0)),
            pl.BlockSpec((1, DN), lambda i: (0, 0)),
            pl.BlockSpec((DF, DN), lambda i: (0, 0)),
            pl.BlockSpec((DN, 1), lambda i: (0, 0)),
        ],
        out_specs=[
            pl.BlockSpec((RB, DN), lambda i: (i, 0)),
            pl.BlockSpec((RB, 1), lambda i: (i, 0)),
        ],
        out_shape=[
            jax.ShapeDtypeStruct((N, DN), jnp.float32),
            jax.ShapeDtypeStruct((N, 1), jnp.float32),
        ],
    )(agg, drug_x, gc_w_rel, gc_b_rel.reshape(1, DN), gc_w_root,
      topk_w.reshape(DN, 1))

    iota = jnp.arange(N, dtype=jnp.float32)
    blo = batch_idx[0::R]
    bhi = batch_idx[R - 1::R]
    jlo = jnp.searchsorted(bhi, blo, side="left").astype(jnp.int32)
    jhi = (jnp.searchsorted(blo, bhi, side="right") - 1).astype(jnp.int32)
    smem_spec = pl.BlockSpec(memory_space=pltpu.SMEM)
    mask = pl.pallas_call(
        _b_body,
        grid=(NBR,),
        in_specs=[
            smem_spec,
            smem_spec,
            pl.BlockSpec((R, 1), lambda i: (i, 0)),
            pl.BlockSpec((R, 1), lambda i: (i, 0)),
            pl.BlockSpec((R, 1), lambda i: (i, 0)),
            pl.BlockSpec((NBR, 1, R), lambda i: (0, 0, 0)),
            pl.BlockSpec((NBR, 1, R), lambda i: (0, 0, 0)),
            pl.BlockSpec((NBR, 1, R), lambda i: (0, 0, 0)),
        ],
        out_specs=pl.BlockSpec((R, 1), lambda i: (i, 0)),
        out_shape=jax.ShapeDtypeStruct((N, 1), jnp.float32),
    )(jlo, jhi, score, batch_idx.reshape(N, 1), iota.reshape(N, 1),
      score.reshape(NBR, 1, R), batch_idx.reshape(NBR, 1, R),
      iota.reshape(NBR, 1, R))

    allx = pl.pallas_call(
        _c_body,
        grid=(NBR,),
        in_specs=[
            smem_spec,
            smem_spec,
            pl.BlockSpec((R, DN), lambda i: (i, 0)),
            pl.BlockSpec((R, 1), lambda i: (i, 0)),
            pl.BlockSpec((R, 1), lambda i: (i, 0)),
            pl.BlockSpec((R, 1), lambda i: (i, 0)),
        ],
        out_specs=pl.BlockSpec((2, GP, DN), lambda i: (0, 0, 0)),
        out_shape=jax.ShapeDtypeStruct((2, GP, DN), jnp.float32),
        scratch_shapes=[
            pltpu.VMEM((GP, 1), jnp.float32),
            pltpu.VMEM((GP, DN), jnp.float32),
            pltpu.VMEM((GP, DN), jnp.float32),
            pltpu.VMEM((GP, DN), jnp.float32),
            pltpu.VMEM((GP, DN), jnp.float32),
        ],
    )(blo, bhi, h, score, mask, batch_idx.reshape(N, 1))

    xcat = jnp.concatenate([x1, x2], axis=0)
    f, resp = pl.pallas_call(
        _d_body,
        out_shape=[
            jax.ShapeDtypeStruct((2 * B, B), jnp.float32),
            jax.ShapeDtypeStruct((B, 1), jnp.float32),
        ],
    )(xcat, em_w1, em_b1.reshape(1, 1024), em_w2, em_b2.reshape(1, B),
      allx[:, :B, :], pm_w[0:B], pm_w[B:].reshape(2, DN, 1),
      pm_b.reshape(1, 1))
    return f[:B], f[B:], resp


# speedup vs baseline: 1.0581x; 1.0581x over previous
import functools

import jax
import jax.numpy as jnp
from jax import lax
from jax.experimental import pallas as pl
from jax.experimental.pallas import tpu as pltpu
from jax.experimental.pallas import tpu_sc as plsc

N = 10000
E = 320000
B = 100
EXPR = 2048
DF = 128
DN = 200
S_BN = 1.0 / (1.0 + 1e-5) ** 0.5

NC, NS = 2, 16
NW = NC * NS
CHUNK = 128
CH0 = 160
CHP = CH0 // 4
NCHROW = NS * CH0
STRIPE = 632
NAGG = NS * STRIPE

RB = 2000
R = 200
NBR = N // R
GP = 104

def _sc_agg_body(drug_hbm, src_hbm, dst_hbm, zeros_hbm, out_hbm,
                 src_v, dst_v, rows_v, agg_sp, sem0, sem1):
    c = lax.axis_index("c")
    s = lax.axis_index("s")
    row0 = s * STRIPE
    pltpu.sync_copy(zeros_hbm, agg_sp.at[pl.ds(row0, STRIPE)])
    plsc.subcore_barrier()

    sems = (sem0, sem1)
    tb = s * CH0

    @pl.when(c == 1)
    def _pipeline():
        for phase in range(4):
            base = pl.multiple_of(tb + phase * CHP, 8)
            pltpu.sync_copy(src_hbm.at[pl.ds(base, CHP)], src_v)
            pltpu.sync_copy(dst_hbm.at[pl.ds(base, CHP)], dst_v)
            pltpu.async_copy(drug_hbm.at[src_v.at[0]], rows_v.at[0], sem0)
            pltpu.async_copy(drug_hbm.at[src_v.at[1]], rows_v.at[1], sem1)

            def pair_body(g, carry):
                for b in range(2):
                    j = 2 * g + b
                    pltpu.make_async_copy(drug_hbm.at[pl.ds(0, CHUNK)],
                                          rows_v.at[b], sems[b]).wait()

                    @pl.when(j + 2 < CHP)
                    def _prefetch(j=j, b=b):
                        pltpu.async_copy(drug_hbm.at[src_v.at[j + 2]],
                                         rows_v.at[b], sems[b])

                    pltpu.sync_copy(rows_v.at[b], agg_sp.at[dst_v.at[j]],
                                    add=True)
                return carry

            lax.fori_loop(0, CHP // 2, pair_body, 0)

    plsc.subcore_barrier()
    pltpu.sync_copy(agg_sp.at[pl.ds(row0, STRIPE)],
                    out_hbm.at[c, pl.ds(row0, STRIPE)])


@functools.lru_cache(maxsize=1)
def _make_sc_agg():
    mesh = plsc.VectorSubcoreMesh(core_axis_name="c", subcore_axis_name="s",
                                  num_cores=NC, num_subcores=NS)
    return pl.kernel(
        _sc_agg_body,
        out_type=jax.ShapeDtypeStruct((NC, NAGG, DF), jnp.float32),
        mesh=mesh,
        scratch_types=[
            pltpu.VMEM((CHP, CHUNK), jnp.int32),
            pltpu.VMEM((CHP, CHUNK), jnp.int32),
            pltpu.VMEM((2, CHUNK, DF), jnp.float32),
            pltpu.VMEM_SHARED((NAGG, DF), jnp.float32),
            pltpu.SemaphoreType.DMA,
            pltpu.SemaphoreType.DMA,
        ],
    )


def _a_body(agg_ref, dx_ref, wrel_ref, brel_ref, wroot_ref, tw_ref,
            h_ref, sc_ref):
    agg = agg_ref[0] + agg_ref[1]
    h = jnp.dot(agg, wrel_ref[...], preferred_element_type=jnp.float32)
    h = h + brel_ref[...]
    h = h + jnp.dot(dx_ref[...], wroot_ref[...],
                    preferred_element_type=jnp.float32)
    h = jnp.maximum(h * S_BN, 0.0)
    h_ref[...] = h
    tw = tw_ref[...]
    inv = lax.rsqrt(jnp.sum(tw * tw))
    sc_ref[...] = jnp.tanh(
        jnp.dot(h, tw, preferred_element_type=jnp.float32) * inv)


def _b_body(jlo_ref, jhi_ref, sr_ref, br_ref, ir_ref,
            sc3_ref, bc3_ref, ic3_ref, mask_ref):
    i = pl.program_id(0)
    sr = sr_ref[...]
    br = br_ref[...]
    ir = ir_ref[...]
    zero = jnp.zeros((R, 1), jnp.float32)

    def col(j, carry):
        rank, n = carry
        scj = sc3_ref[j]
        bcj = bc3_ref[j]
        icj = ic3_ref[j]
        same = br == bcj
        better = (scj > sr) | ((scj == sr) & (icj < ir))
        rank = rank + jnp.sum(jnp.where(same & better, 1.0, 0.0),
                              axis=1, keepdims=True)
        n = n + jnp.sum(jnp.where(same, 1.0, 0.0), axis=1, keepdims=True)
        return rank, n

    rank, n = lax.fori_loop(jlo_ref[i], jhi_ref[i] + 1, col, (zero, zero))
    kper = jnp.ceil(0.8 * n)
    mask_ref[...] = jnp.where(rank < kper, 1.0, 0.0)


def _c_body(blo_ref, bhi_ref, h_ref, sc_ref, mk_ref, br_ref, out_ref,
            cnt_ref, sum1_ref, max1_ref, sum2_ref, max2_ref):
    i = pl.program_id(0)

    @pl.when(i == 0)
    def _init():
        cnt_ref[...] = jnp.zeros((GP, 1), jnp.float32)
        sum1_ref[...] = jnp.zeros((GP, DN), jnp.float32)
        max1_ref[...] = jnp.full((GP, DN), -1e30, jnp.float32)
        sum2_ref[...] = jnp.zeros((GP, DN), jnp.float32)
        max2_ref[...] = jnp.full((GP, DN), -1e30, jnp.float32)

    hb = h_ref[...]
    scb = sc_ref[...]
    mkb = mk_ref[...]
    bb = br_ref[...]
    scaled = hb * scb
    g_lo = blo_ref[i]
    g_hi = bhi_ref[i]

    def per_g(g, carry):
        sel = bb == g
        s1 = jnp.sum(jnp.where(sel, hb, 0.0), axis=0, keepdims=True)
        m1 = jnp.max(jnp.where(sel, hb, -1e30), axis=0, keepdims=True)
        selm = sel & (mkb > 0.5)
        s2 = jnp.sum(jnp.where(selm, scaled, 0.0), axis=0, keepdims=True)
        m2 = jnp.max(jnp.where(selm, scaled, -1e30), axis=0, keepdims=True)
        cg = jnp.sum(jnp.where(sel, 1.0, 0.0), axis=0, keepdims=True)
        cnt_ref[pl.ds(g, 1), :] = cnt_ref[pl.ds(g, 1), :] + cg
        sum1_ref[pl.ds(g, 1), :] = sum1_ref[pl.ds(g, 1), :] + s1
        max1_ref[pl.ds(g, 1), :] = jnp.maximum(max1_ref[pl.ds(g, 1), :], m1)
        sum2_ref[pl.ds(g, 1), :] = sum2_ref[pl.ds(g, 1), :] + s2
        max2_ref[pl.ds(g, 1), :] = jnp.maximum(max2_ref[pl.ds(g, 1), :], m2)
        return carry

    lax.fori_loop(g_lo, g_hi + 1, per_g, 0)

    @pl.when(i == NBR - 1)
    def _fin():
        cnt = cnt_ref[...]
        kper = jnp.ceil(0.8 * cnt)
        gmp1 = jnp.where(cnt > 0, max1_ref[...], 0.0)
        gap1 = sum1_ref[...] / jnp.maximum(cnt, 1.0)
        gmp2 = jnp.where(kper > 0, max2_ref[...], 0.0)
        gap2 = sum2_ref[...] / jnp.maximum(kper, 1.0)
        out_ref[0] = jnp.maximum(gmp1 + 3.0 * gmp2, 0.0)
        out_ref[1] = jnp.maximum(gap1 + 3.0 * gap2, 0.0)


def _d_body(x_ref, w1_ref, b1_ref, w2_ref, b2_ref, ax_ref,
            pwf_ref, pwa_ref, pb_ref, f_ref, r_ref):
    x = x_ref[...] * S_BN
    h1 = jnp.dot(x, w1_ref[...], preferred_element_type=jnp.float32)
    h1 = jnp.maximum((h1 + b1_ref[...]) * S_BN, 0.0)
    f = jnp.dot(h1, w2_ref[...], preferred_element_type=jnp.float32)
    f = jnp.maximum((f + b2_ref[...]) * S_BN, 0.0)
    f_ref[...] = f
    f1 = f[0:B, :]
    r = jnp.dot(f1, pwf_ref[...], preferred_element_type=jnp.float32)
    r = r + jnp.dot(ax_ref[0], pwa_ref[0],
                    preferred_element_type=jnp.float32)
    r = r + jnp.dot(ax_ref[1], pwa_ref[1],
                    preferred_element_type=jnp.float32)
    r_ref[...] = r + pb_ref[...]


def kernel(x1, x2, batch_idx, edge_attr, edge_index, drug_x,
           em_w1, em_b1, em_w2, em_b2, gc_w_rel, gc_b_rel, gc_w_root,
           topk_w, pm_w, pm_b):
    del edge_attr
    batch_idx = batch_idx.astype(jnp.int32)
    edge_index = edge_index.astype(jnp.int32)

    npad = NCHROW * CHUNK - E
    src_p = jnp.concatenate(
        [edge_index[0], jnp.full((npad,), N, jnp.int32)]
    ).reshape(NCHROW, CHUNK)
    dst_p = jnp.concatenate(
        [edge_index[1], jnp.zeros((npad,), jnp.int32)]
    ).reshape(NCHROW, CHUNK)
    drug_pad = jnp.concatenate(
        [drug_x, jnp.zeros((16, DF), jnp.float32)], axis=0)
    zeros_hbm = jnp.zeros((STRIPE, DF), jnp.float32)
    agg = _make_sc_agg()(drug_pad, src_p, dst_p, zeros_hbm)[:, :N, :]

    h, score = pl.pallas_call(
        _a_body,
        grid=(N // RB,),
        in_specs=[
            pl.BlockSpec((NC, RB, DF), lambda i: (0, i, 0)),
            pl.BlockSpec((RB, DF), lambda i: (i, 0)),
            pl.BlockSpec((DF, DN), lambda i: (0, 0)),
            pl.BlockSpec((1, DN), lambda i: (0, 0)),
            pl.BlockSpec((DF, DN), lambda i: (0, 0)),
            pl.BlockSpec((DN, 1), lambda i: (0, 0)),
        ],
        out_specs=[
            pl.BlockSpec((RB, DN), lambda i: (i, 0)),
            pl.BlockSpec((RB, 1), lambda i: (i, 0)),
        ],
        out_shape=[
            jax.ShapeDtypeStruct((N, DN), jnp.float32),
            jax.ShapeDtypeStruct((N, 1), jnp.float32),
        ],
    )(agg, drug_x, gc_w_rel, gc_b_rel.reshape(1, DN), gc_w_root,
      topk_w.reshape(DN, 1))

    iota = jnp.arange(N, dtype=jnp.float32)
    blo = batch_idx[0::R]
    bhi = batch_idx[R - 1::R]
    jlo = jnp.searchsorted(bhi, blo, side="left").astype(jnp.int32)
    jhi = (jnp.searchsorted(blo, bhi, side="right") - 1).astype(jnp.int32)
    smem_spec = pl.BlockSpec(memory_space=pltpu.SMEM)
    mask = pl.pallas_call(
        _b_body,
        grid=(NBR,),
        in_specs=[
            smem_spec,
            smem_spec,
            pl.BlockSpec((R, 1), lambda i: (i, 0)),
            pl.BlockSpec((R, 1), lambda i: (i, 0)),
            pl.BlockSpec((R, 1), lambda i: (i, 0)),
            pl.BlockSpec((NBR, 1, R), lambda i: (0, 0, 0)),
            pl.BlockSpec((NBR, 1, R), lambda i: (0, 0, 0)),
            pl.BlockSpec((NBR, 1, R), lambda i: (0, 0, 0)),
        ],
        out_specs=pl.BlockSpec((R, 1), lambda i: (i, 0)),
        out_shape=jax.ShapeDtypeStruct((N, 1), jnp.float32),
    )(jlo, jhi, score, batch_idx.reshape(N, 1), iota.reshape(N, 1),
      score.reshape(NBR, 1, R), batch_idx.reshape(NBR, 1, R),
      iota.reshape(NBR, 1, R))

    allx = pl.pallas_call(
        _c_body,
        grid=(NBR,),
        in_specs=[
            smem_spec,
            smem_spec,
            pl.BlockSpec((R, DN), lambda i: (i, 0)),
            pl.BlockSpec((R, 1), lambda i: (i, 0)),
            pl.BlockSpec((R, 1), lambda i: (i, 0)),
            pl.BlockSpec((R, 1), lambda i: (i, 0)),
        ],
        out_specs=pl.BlockSpec((2, GP, DN), lambda i: (0, 0, 0)),
        out_shape=jax.ShapeDtypeStruct((2, GP, DN), jnp.float32),
        scratch_shapes=[
            pltpu.VMEM((GP, 1), jnp.float32),
            pltpu.VMEM((GP, DN), jnp.float32),
            pltpu.VMEM((GP, DN), jnp.float32),
            pltpu.VMEM((GP, DN), jnp.float32),
            pltpu.VMEM((GP, DN), jnp.float32),
        ],
    )(blo, bhi, h, score, mask, batch_idx.reshape(N, 1))

    xcat = jnp.concatenate([x1, x2], axis=0)
    f, resp = pl.pallas_call(
        _d_body,
        out_shape=[
            jax.ShapeDtypeStruct((2 * B, B), jnp.float32),
            jax.ShapeDtypeStruct((B, 1), jnp.float32),
        ],
    )(xcat, em_w1, em_b1.reshape(1, 1024), em_w2, em_b2.reshape(1, B),
      allx[:, :B, :], pm_w[0:B], pm_w[B:].reshape(2, DN, 1),
      pm_b.reshape(1, 1))
    return f[:B], f[B:], resp


# speedup vs baseline: 1.1653x; 1.1012x over previous
import functools

import jax
import jax.numpy as jnp
from jax import lax
from jax.experimental import pallas as pl
from jax.experimental.pallas import tpu as pltpu
from jax.experimental.pallas import tpu_sc as plsc

N = 10000
E = 320000
B = 100
EXPR = 2048
DF = 128
DN = 200
S_BN = 1.0 / (1.0 + 1e-5) ** 0.5

NC, NS = 2, 16
NW = NC * NS
CHUNK = 128
CH = 80
CH0 = 112
CH1 = 2 * CH - CH0
CHH0 = CH0 // 2
NCHROW = NS * (CH0 + CH1)
E_PAD = NW * CH * CHUNK
NPAD = N + 16
STRIPE = 632
NAGG = NS * STRIPE

RB = 2000
R = 200
NBR = N // R
GP = 104

def _sc_agg_body(drug_hbm, src_hbm, dst_hbm, zeros_hbm, out_hbm,
                 src_v, dst_v, rows_v, agg_sp, sem0, sem1):
    c = lax.axis_index("c")
    s = lax.axis_index("s")
    row0 = s * STRIPE
    pltpu.sync_copy(zeros_hbm, agg_sp.at[pl.ds(row0, STRIPE)])
    plsc.subcore_barrier()

    tb = jnp.where(c == 0, s * CH0, NS * CH0 + s * CH1)
    nch = jnp.where(c == 0, CH0, CH1)
    chh = nch // 2
    sems = (sem0, sem1)
    for phase in range(2):
        base = pl.multiple_of(tb + phase * chh, 8)
        pltpu.sync_copy(src_hbm.at[pl.ds(base, CHH0)], src_v)
        pltpu.sync_copy(dst_hbm.at[pl.ds(base, CHH0)], dst_v)
        pltpu.async_copy(drug_hbm.at[src_v.at[0]], rows_v.at[0], sem0)
        pltpu.async_copy(drug_hbm.at[src_v.at[1]], rows_v.at[1], sem1)

        def pair_body(g, carry):
            for b in range(2):
                j = 2 * g + b
                pltpu.make_async_copy(drug_hbm.at[pl.ds(0, CHUNK)],
                                      rows_v.at[b], sems[b]).wait()

                @pl.when(j + 2 < chh)
                def _prefetch(j=j, b=b):
                    pltpu.async_copy(drug_hbm.at[src_v.at[j + 2]],
                                     rows_v.at[b], sems[b])

                pltpu.sync_copy(rows_v.at[b], agg_sp.at[dst_v.at[j]],
                                add=True)
            return carry

        lax.fori_loop(0, chh // 2, pair_body, 0)
    plsc.subcore_barrier()
    pltpu.sync_copy(agg_sp.at[pl.ds(row0, STRIPE)],
                    out_hbm.at[c, pl.ds(row0, STRIPE)])


@functools.lru_cache(maxsize=1)
def _make_sc_agg():
    mesh = plsc.VectorSubcoreMesh(core_axis_name="c", subcore_axis_name="s",
                                  num_cores=NC, num_subcores=NS)
    return pl.kernel(
        _sc_agg_body,
        out_type=jax.ShapeDtypeStruct((NC, NAGG, DF), jnp.float32),
        mesh=mesh,
        scratch_types=[
            pltpu.VMEM((CHH0, CHUNK), jnp.int32),
            pltpu.VMEM((CHH0, CHUNK), jnp.int32),
            pltpu.VMEM((2, CHUNK, DF), jnp.float32),
            pltpu.VMEM_SHARED((NAGG, DF), jnp.float32),
            pltpu.SemaphoreType.DMA,
            pltpu.SemaphoreType.DMA,
        ],
    )


def _a_body(agg_ref, dx_ref, wrel_ref, brel_ref, wroot_ref, tw_ref,
            h_ref, sc_ref):
    agg = agg_ref[0] + agg_ref[1]
    h = jnp.dot(agg, wrel_ref[...], preferred_element_type=jnp.float32)
    h = h + brel_ref[...]
    h = h + jnp.dot(dx_ref[...], wroot_ref[...],
                    preferred_element_type=jnp.float32)
    h = jnp.maximum(h * S_BN, 0.0)
    h_ref[...] = h
    tw = tw_ref[...]
    inv = lax.rsqrt(jnp.sum(tw * tw))
    sc_ref[...] = jnp.tanh(
        jnp.dot(h, tw, preferred_element_type=jnp.float32) * inv)


def _b_body(jlo_ref, jhi_ref, sr_ref, br_ref, ir_ref,
            sc3_ref, bc3_ref, ic3_ref, mask_ref):
    i = pl.program_id(0)
    sr = sr_ref[...]
    br = br_ref[...]
    ir = ir_ref[...]
    zero = jnp.zeros((R, 1), jnp.float32)

    def col(j, carry):
        rank, n = carry
        scj = sc3_ref[j]
        bcj = bc3_ref[j]
        icj = ic3_ref[j]
        same = br == bcj
        better = (scj > sr) | ((scj == sr) & (icj < ir))
        rank = rank + jnp.sum(jnp.where(same & better, 1.0, 0.0),
                              axis=1, keepdims=True)
        n = n + jnp.sum(jnp.where(same, 1.0, 0.0), axis=1, keepdims=True)
        return rank, n

    rank, n = lax.fori_loop(jlo_ref[i], jhi_ref[i] + 1, col, (zero, zero))
    kper = jnp.ceil(0.8 * n)
    mask_ref[...] = jnp.where(rank < kper, 1.0, 0.0)


def _c_body(blo_ref, bhi_ref, h_ref, sc_ref, mk_ref, br_ref, out_ref,
            cnt_ref, sum1_ref, max1_ref, sum2_ref, max2_ref):
    i = pl.program_id(0)

    @pl.when(i == 0)
    def _init():
        cnt_ref[...] = jnp.zeros((GP, 1), jnp.float32)
        sum1_ref[...] = jnp.zeros((GP, DN), jnp.float32)
        max1_ref[...] = jnp.full((GP, DN), -1e30, jnp.float32)
        sum2_ref[...] = jnp.zeros((GP, DN), jnp.float32)
        max2_ref[...] = jnp.full((GP, DN), -1e30, jnp.float32)

    hb = h_ref[...]
    scb = sc_ref[...]
    mkb = mk_ref[...]
    bb = br_ref[...]
    scaled = hb * scb
    g_lo = blo_ref[i]
    g_hi = bhi_ref[i]

    def per_g(g, carry):
        sel = bb == g
        s1 = jnp.sum(jnp.where(sel, hb, 0.0), axis=0, keepdims=True)
        m1 = jnp.max(jnp.where(sel, hb, -1e30), axis=0, keepdims=True)
        selm = sel & (mkb > 0.5)
        s2 = jnp.sum(jnp.where(selm, scaled, 0.0), axis=0, keepdims=True)
        m2 = jnp.max(jnp.where(selm, scaled, -1e30), axis=0, keepdims=True)
        cg = jnp.sum(jnp.where(sel, 1.0, 0.0), axis=0, keepdims=True)
        cnt_ref[pl.ds(g, 1), :] = cnt_ref[pl.ds(g, 1), :] + cg
        sum1_ref[pl.ds(g, 1), :] = sum1_ref[pl.ds(g, 1), :] + s1
        max1_ref[pl.ds(g, 1), :] = jnp.maximum(max1_ref[pl.ds(g, 1), :], m1)
        sum2_ref[pl.ds(g, 1), :] = sum2_ref[pl.ds(g, 1), :] + s2
        max2_ref[pl.ds(g, 1), :] = jnp.maximum(max2_ref[pl.ds(g, 1), :], m2)
        return carry

    lax.fori_loop(g_lo, g_hi + 1, per_g, 0)

    @pl.when(i == NBR - 1)
    def _fin():
        cnt = cnt_ref[...]
        kper = jnp.ceil(0.8 * cnt)
        gmp1 = jnp.where(cnt > 0, max1_ref[...], 0.0)
        gap1 = sum1_ref[...] / jnp.maximum(cnt, 1.0)
        gmp2 = jnp.where(kper > 0, max2_ref[...], 0.0)
        gap2 = sum2_ref[...] / jnp.maximum(kper, 1.0)
        out_ref[0] = jnp.maximum(gmp1 + 3.0 * gmp2, 0.0)
        out_ref[1] = jnp.maximum(gap1 + 3.0 * gap2, 0.0)


def _d_body(x_ref, w1_ref, b1_ref, w2_ref, b2_ref, ax_ref,
            pwf_ref, pwa_ref, pb_ref, f_ref, r_ref):
    x = x_ref[...] * S_BN
    h1 = jnp.dot(x, w1_ref[...], preferred_element_type=jnp.float32)
    h1 = jnp.maximum((h1 + b1_ref[...]) * S_BN, 0.0)
    f = jnp.dot(h1, w2_ref[...], preferred_element_type=jnp.float32)
    f = jnp.maximum((f + b2_ref[...]) * S_BN, 0.0)
    f_ref[...] = f
    f1 = f[0:B, :]
    r = jnp.dot(f1, pwf_ref[...], preferred_element_type=jnp.float32)
    r = r + jnp.dot(ax_ref[0], pwa_ref[0],
                    preferred_element_type=jnp.float32)
    r = r + jnp.dot(ax_ref[1], pwa_ref[1],
                    preferred_element_type=jnp.float32)
    r_ref[...] = r + pb_ref[...]


def kernel(x1, x2, batch_idx, edge_attr, edge_index, drug_x,
           em_w1, em_b1, em_w2, em_b2, gc_w_rel, gc_b_rel, gc_w_root,
           topk_w, pm_w, pm_b):
    del edge_attr
    batch_idx = batch_idx.astype(jnp.int32)
    edge_index = edge_index.astype(jnp.int32)

    npad = (NCHROW + CHH0) * CHUNK - E
    src_p = jnp.concatenate(
        [edge_index[0], jnp.full((npad,), N, jnp.int32)]
    ).reshape(NCHROW + CHH0, CHUNK)
    dst_p = jnp.concatenate(
        [edge_index[1], jnp.zeros((npad,), jnp.int32)]
    ).reshape(NCHROW + CHH0, CHUNK)
    drug_pad = jnp.concatenate(
        [drug_x, jnp.zeros((NPAD - N, DF), jnp.float32)], axis=0)
    zeros_hbm = jnp.zeros((STRIPE, DF), jnp.float32)
    agg = _make_sc_agg()(drug_pad, src_p, dst_p, zeros_hbm)

    h, score = pl.pallas_call(
        _a_body,
        grid=(N // RB,),
        in_specs=[
            pl.BlockSpec((NC, RB, DF), lambda i: (0, i, 0)),
            pl.BlockSpec((RB, DF), lambda i: (i, 0)),
            pl.BlockSpec((DF, DN), lambda i: (0, 0)),
            pl.BlockSpec((1, DN), lambda i: (0, 0)),
            pl.BlockSpec((DF, DN), lambda i: (0, 0)),
            pl.BlockSpec((DN, 1), lambda i: (0, 0)),
        ],
        out_specs=[
            pl.BlockSpec((RB, DN), lambda i: (i, 0)),
            pl.BlockSpec((RB, 1), lambda i: (i, 0)),
        ],
        out_shape=[
            jax.ShapeDtypeStruct((N, DN), jnp.float32),
            jax.ShapeDtypeStruct((N, 1), jnp.float32),
        ],
    )(agg, drug_x, gc_w_rel, gc_b_rel.reshape(1, DN), gc_w_root,
      topk_w.reshape(DN, 1))

    iota = jnp.arange(N, dtype=jnp.float32)
    blo = batch_idx[0::R]
    bhi = batch_idx[R - 1::R]
    jlo = jnp.searchsorted(bhi, blo, side="left").astype(jnp.int32)
    jhi = (jnp.searchsorted(blo, bhi, side="right") - 1).astype(jnp.int32)
    smem_spec = pl.BlockSpec(memory_space=pltpu.SMEM)
    mask = pl.pallas_call(
        _b_body,
        grid=(NBR,),
        in_specs=[
            smem_spec,
            smem_spec,
            pl.BlockSpec((R, 1), lambda i: (i, 0)),
            pl.BlockSpec((R, 1), lambda i: (i, 0)),
            pl.BlockSpec((R, 1), lambda i: (i, 0)),
            pl.BlockSpec((NBR, 1, R), lambda i: (0, 0, 0)),
            pl.BlockSpec((NBR, 1, R), lambda i: (0, 0, 0)),
            pl.BlockSpec((NBR, 1, R), lambda i: (0, 0, 0)),
        ],
        out_specs=pl.BlockSpec((R, 1), lambda i: (i, 0)),
        out_shape=jax.ShapeDtypeStruct((N, 1), jnp.float32),
    )(jlo, jhi, score, batch_idx.reshape(N, 1), iota.reshape(N, 1),
      score.reshape(NBR, 1, R), batch_idx.reshape(NBR, 1, R),
      iota.reshape(NBR, 1, R))

    allx = pl.pallas_call(
        _c_body,
        grid=(NBR,),
        in_specs=[
            smem_spec,
            smem_spec,
            pl.BlockSpec((R, DN), lambda i: (i, 0)),
            pl.BlockSpec((R, 1), lambda i: (i, 0)),
            pl.BlockSpec((R, 1), lambda i: (i, 0)),
            pl.BlockSpec((R, 1), lambda i: (i, 0)),
        ],
        out_specs=pl.BlockSpec((2, GP, DN), lambda i: (0, 0, 0)),
        out_shape=jax.ShapeDtypeStruct((2, GP, DN), jnp.float32),
        scratch_shapes=[
            pltpu.VMEM((GP, 1), jnp.float32),
            pltpu.VMEM((GP, DN), jnp.float32),
            pltpu.VMEM((GP, DN), jnp.float32),
            pltpu.VMEM((GP, DN), jnp.float32),
            pltpu.VMEM((GP, DN), jnp.float32),
        ],
    )(blo, bhi, h, score, mask, batch_idx.reshape(N, 1))

    xcat = jnp.concatenate([x1, x2], axis=0)
    f, resp = pl.pallas_call(
        _d_body,
        out_shape=[
            jax.ShapeDtypeStruct((2 * B, B), jnp.float32),
            jax.ShapeDtypeStruct((B, 1), jnp.float32),
        ],
    )(xcat, em_w1, em_b1.reshape(1, 1024), em_w2, em_b2.reshape(1, B),
      allx[:, :B, :], pm_w[0:B], pm_w[B:].reshape(2, DN, 1),
      pm_b.reshape(1, 1))
    return f[:B], f[B:], resp


# speedup vs baseline: 1.1986x; 1.0286x over previous
import functools

import jax
import jax.numpy as jnp
from jax import lax
from jax.experimental import pallas as pl
from jax.experimental.pallas import tpu as pltpu
from jax.experimental.pallas import tpu_sc as plsc

N = 10000
E = 320000
B = 100
EXPR = 2048
DF = 128
DN = 200
S_BN = 1.0 / (1.0 + 1e-5) ** 0.5

NC, NS = 2, 16
NW = NC * NS
CHUNK = 128
CH = 80
CH0 = 112
CH1 = 2 * CH - CH0
CHH0 = CH0 // 2
NCHROW = NS * (CH0 + CH1)
E_PAD = NW * CH * CHUNK
NPAD = N + 16
STRIPE = 632
NAGG = NS * STRIPE

RB = 2000
R = 200
NBR = N // R
GP = 104

def _sc_agg_body(drug_hbm, src_hbm, dst_hbm, zeros_hbm, out_hbm,
                 src_v, dst_v, rows_v, agg_sp, sem0, sem1):
    c = lax.axis_index("c")
    s = lax.axis_index("s")
    row0 = s * STRIPE
    pltpu.sync_copy(zeros_hbm, agg_sp.at[pl.ds(row0, STRIPE)])
    plsc.subcore_barrier()

    tb = jnp.where(c == 0, s * CH0, NS * CH0 + s * CH1)
    nch = jnp.where(c == 0, CH0, CH1)
    chh = nch // 2
    sems = (sem0, sem1)
    for phase in range(2):
        base = pl.multiple_of(tb + phase * chh, 8)
        pltpu.sync_copy(src_hbm.at[pl.ds(base, CHH0)], src_v)
        pltpu.sync_copy(dst_hbm.at[pl.ds(base, CHH0)], dst_v)
        pltpu.async_copy(drug_hbm.at[src_v.at[0]], rows_v.at[0], sem0)
        pltpu.async_copy(drug_hbm.at[src_v.at[1]], rows_v.at[1], sem1)

        def pair_body(g, carry):
            for b in range(2):
                j = 2 * g + b
                pltpu.make_async_copy(drug_hbm.at[pl.ds(0, CHUNK)],
                                      rows_v.at[b], sems[b]).wait()

                @pl.when(j + 2 < chh)
                def _prefetch(j=j, b=b):
                    pltpu.async_copy(drug_hbm.at[src_v.at[j + 2]],
                                     rows_v.at[b], sems[b])

                pltpu.sync_copy(rows_v.at[b], agg_sp.at[dst_v.at[j]],
                                add=True)
            return carry

        lax.fori_loop(0, chh // 2, pair_body, 0)
    plsc.subcore_barrier()
    pltpu.sync_copy(agg_sp.at[pl.ds(row0, STRIPE)],
                    out_hbm.at[c, pl.ds(row0, STRIPE)])


@functools.lru_cache(maxsize=1)
def _make_sc_agg():
    mesh = plsc.VectorSubcoreMesh(core_axis_name="c", subcore_axis_name="s",
                                  num_cores=NC, num_subcores=NS)
    return pl.kernel(
        _sc_agg_body,
        out_type=jax.ShapeDtypeStruct((NC, NAGG, DF), jnp.float32),
        mesh=mesh,
        scratch_types=[
            pltpu.VMEM((CHH0, CHUNK), jnp.int32),
            pltpu.VMEM((CHH0, CHUNK), jnp.int32),
            pltpu.VMEM((2, CHUNK, DF), jnp.float32),
            pltpu.VMEM_SHARED((NAGG, DF), jnp.float32),
            pltpu.SemaphoreType.DMA,
            pltpu.SemaphoreType.DMA,
        ],
    )


def _a_body(agg_ref, dx_ref, wrel_ref, brel_ref, wroot_ref, tw_ref,
            h_ref, sc_ref):
    agg = agg_ref[0] + agg_ref[1]
    h = jnp.dot(agg, wrel_ref[...], preferred_element_type=jnp.float32)
    h = h + brel_ref[...]
    h = h + jnp.dot(dx_ref[...], wroot_ref[...],
                    preferred_element_type=jnp.float32)
    h = jnp.maximum(h * S_BN, 0.0)
    h_ref[...] = h
    tw = tw_ref[...]
    inv = lax.rsqrt(jnp.sum(tw * tw))
    sc_ref[...] = jnp.tanh(
        jnp.dot(h, tw, preferred_element_type=jnp.float32) * inv)


def _bc_body(jlo_ref, jhi_ref, blo_ref, bhi_ref, h_ref, sr_ref, ir_ref,
             br_ref, sc3_ref, bc3_ref, ic3_ref, out_ref,
             cnt_ref, sum1_ref, max1_ref, sum2_ref, max2_ref):
    i = pl.program_id(0)

    @pl.when(i == 0)
    def _init():
        cnt_ref[...] = jnp.zeros((GP, 1), jnp.float32)
        sum1_ref[...] = jnp.zeros((GP, DN), jnp.float32)
        max1_ref[...] = jnp.full((GP, DN), -1e30, jnp.float32)
        sum2_ref[...] = jnp.zeros((GP, DN), jnp.float32)
        max2_ref[...] = jnp.full((GP, DN), -1e30, jnp.float32)

    sr = sr_ref[...]
    br = br_ref[...]
    ir = ir_ref[...]
    zero = jnp.zeros((R, 1), jnp.float32)

    def col(j, carry):
        rank, n = carry
        scj = sc3_ref[j]
        bcj = bc3_ref[j]
        icj = ic3_ref[j]
        same = br == bcj
        better = (scj > sr) | ((scj == sr) & (icj < ir))
        rank = rank + jnp.sum(jnp.where(same & better, 1.0, 0.0),
                              axis=1, keepdims=True)
        n = n + jnp.sum(jnp.where(same, 1.0, 0.0), axis=1, keepdims=True)
        return rank, n

    rank, n = lax.fori_loop(jlo_ref[i], jhi_ref[i] + 1, col, (zero, zero))
    kper_node = jnp.ceil(0.8 * n)
    mkb = rank < kper_node

    hb = h_ref[...]
    bb = br_ref[...]
    scaled = hb * sr
    g_lo = blo_ref[i]
    g_hi = bhi_ref[i]

    def per_g(g, carry):
        sel = bb == g
        s1 = jnp.sum(jnp.where(sel, hb, 0.0), axis=0, keepdims=True)
        m1 = jnp.max(jnp.where(sel, hb, -1e30), axis=0, keepdims=True)
        selm = sel & mkb
        s2 = jnp.sum(jnp.where(selm, scaled, 0.0), axis=0, keepdims=True)
        m2 = jnp.max(jnp.where(selm, scaled, -1e30), axis=0, keepdims=True)
        cg = jnp.sum(jnp.where(sel, 1.0, 0.0), axis=0, keepdims=True)
        cnt_ref[pl.ds(g, 1), :] = cnt_ref[pl.ds(g, 1), :] + cg
        sum1_ref[pl.ds(g, 1), :] = sum1_ref[pl.ds(g, 1), :] + s1
        max1_ref[pl.ds(g, 1), :] = jnp.maximum(max1_ref[pl.ds(g, 1), :], m1)
        sum2_ref[pl.ds(g, 1), :] = sum2_ref[pl.ds(g, 1), :] + s2
        max2_ref[pl.ds(g, 1), :] = jnp.maximum(max2_ref[pl.ds(g, 1), :], m2)
        return carry

    lax.fori_loop(g_lo, g_hi + 1, per_g, 0)

    @pl.when(i == NBR - 1)
    def _fin():
        cnt = cnt_ref[...]
        kper = jnp.ceil(0.8 * cnt)
        gmp1 = jnp.where(cnt > 0, max1_ref[...], 0.0)
        gap1 = sum1_ref[...] / jnp.maximum(cnt, 1.0)
        gmp2 = jnp.where(kper > 0, max2_ref[...], 0.0)
        gap2 = sum2_ref[...] / jnp.maximum(kper, 1.0)
        out_ref[0] = jnp.maximum(gmp1 + 3.0 * gmp2, 0.0)
        out_ref[1] = jnp.maximum(gap1 + 3.0 * gap2, 0.0)


def _d_body(x_ref, w1_ref, b1_ref, w2_ref, b2_ref, ax_ref,
            pwf_ref, pwa_ref, pb_ref, f_ref, r_ref):
    x = x_ref[...] * S_BN
    h1 = jnp.dot(x, w1_ref[...], preferred_element_type=jnp.float32)
    h1 = jnp.maximum((h1 + b1_ref[...]) * S_BN, 0.0)
    f = jnp.dot(h1, w2_ref[...], preferred_element_type=jnp.float32)
    f = jnp.maximum((f + b2_ref[...]) * S_BN, 0.0)
    f_ref[...] = f
    f1 = f[0:B, :]
    r = jnp.dot(f1, pwf_ref[...], preferred_element_type=jnp.float32)
    r = r + jnp.dot(ax_ref[0], pwa_ref[0],
                    preferred_element_type=jnp.float32)
    r = r + jnp.dot(ax_ref[1], pwa_ref[1],
                    preferred_element_type=jnp.float32)
    r_ref[...] = r + pb_ref[...]


def kernel(x1, x2, batch_idx, edge_attr, edge_index, drug_x,
           em_w1, em_b1, em_w2, em_b2, gc_w_rel, gc_b_rel, gc_w_root,
           topk_w, pm_w, pm_b):
    del edge_attr
    batch_idx = batch_idx.astype(jnp.int32)
    edge_index = edge_index.astype(jnp.int32)

    npad = (NCHROW + CHH0) * CHUNK - E
    src_p = jnp.concatenate(
        [edge_index[0], jnp.full((npad,), N, jnp.int32)]
    ).reshape(NCHROW + CHH0, CHUNK)
    dst_p = jnp.concatenate(
        [edge_index[1], jnp.zeros((npad,), jnp.int32)]
    ).reshape(NCHROW + CHH0, CHUNK)
    drug_pad = jnp.concatenate(
        [drug_x, jnp.zeros((NPAD - N, DF), jnp.float32)], axis=0)
    zeros_hbm = jnp.zeros((STRIPE, DF), jnp.float32)
    agg = _make_sc_agg()(drug_pad, src_p, dst_p, zeros_hbm)

    h, score = pl.pallas_call(
        _a_body,
        grid=(N // RB,),
        in_specs=[
            pl.BlockSpec((NC, RB, DF), lambda i: (0, i, 0)),
            pl.BlockSpec((RB, DF), lambda i: (i, 0)),
            pl.BlockSpec((DF, DN), lambda i: (0, 0)),
            pl.BlockSpec((1, DN), lambda i: (0, 0)),
            pl.BlockSpec((DF, DN), lambda i: (0, 0)),
            pl.BlockSpec((DN, 1), lambda i: (0, 0)),
        ],
        out_specs=[
            pl.BlockSpec((RB, DN), lambda i: (i, 0)),
            pl.BlockSpec((RB, 1), lambda i: (i, 0)),
        ],
        out_shape=[
            jax.ShapeDtypeStruct((N, DN), jnp.float32),
            jax.ShapeDtypeStruct((N, 1), jnp.float32),
        ],
    )(agg, drug_x, gc_w_rel, gc_b_rel.reshape(1, DN), gc_w_root,
      topk_w.reshape(DN, 1))

    iota = jnp.arange(N, dtype=jnp.float32)
    blo = batch_idx[0::R]
    bhi = batch_idx[R - 1::R]
    jlo = jnp.searchsorted(bhi, blo, side="left").astype(jnp.int32)
    jhi = (jnp.searchsorted(blo, bhi, side="right") - 1).astype(jnp.int32)
    smem_spec = pl.BlockSpec(memory_space=pltpu.SMEM)
    allx = pl.pallas_call(
        _bc_body,
        grid=(NBR,),
        in_specs=[
            smem_spec,
            smem_spec,
            smem_spec,
            smem_spec,
            pl.BlockSpec((R, DN), lambda i: (i, 0)),
            pl.BlockSpec((R, 1), lambda i: (i, 0)),
            pl.BlockSpec((R, 1), lambda i: (i, 0)),
            pl.BlockSpec((R, 1), lambda i: (i, 0)),
            pl.BlockSpec((NBR, 1, R), lambda i: (0, 0, 0)),
            pl.BlockSpec((NBR, 1, R), lambda i: (0, 0, 0)),
            pl.BlockSpec((NBR, 1, R), lambda i: (0, 0, 0)),
        ],
        out_specs=pl.BlockSpec((2, GP, DN), lambda i: (0, 0, 0)),
        out_shape=jax.ShapeDtypeStruct((2, GP, DN), jnp.float32),
        scratch_shapes=[
            pltpu.VMEM((GP, 1), jnp.float32),
            pltpu.VMEM((GP, DN), jnp.float32),
            pltpu.VMEM((GP, DN), jnp.float32),
            pltpu.VMEM((GP, DN), jnp.float32),
            pltpu.VMEM((GP, DN), jnp.float32),
        ],
    )(jlo, jhi, blo, bhi, h, score, iota.reshape(N, 1),
      batch_idx.reshape(N, 1), score.reshape(NBR, 1, R),
      batch_idx.reshape(NBR, 1, R), iota.reshape(NBR, 1, R))

    xcat = jnp.concatenate([x1, x2], axis=0)
    f, resp = pl.pallas_call(
        _d_body,
        out_shape=[
            jax.ShapeDtypeStruct((2 * B, B), jnp.float32),
            jax.ShapeDtypeStruct((B, 1), jnp.float32),
        ],
    )(xcat, em_w1, em_b1.reshape(1, 1024), em_w2, em_b2.reshape(1, B),
      allx[:, :B, :], pm_w[0:B], pm_w[B:].reshape(2, DN, 1),
      pm_b.reshape(1, 1))
    return f[:B], f[B:], resp
